# baseline (speedup 1.0000x reference)
import functools

import jax
import jax.numpy as jnp
from jax import lax
from jax.experimental import pallas as pl
from jax.experimental.pallas import tpu as pltpu
from jax.experimental.pallas import tpu_sc as plsc

N = 10000
NE = 10000
NNZ = 320000
D = 128
NC = 2
NS = 16
NW = NC * NS
CHUNK = NNZ // NW
KW = 80
NWIN = CHUNK // KW
NP = 10240
RPT = NP // NS
HR = NP // D

BR = 1000
GRID = N // BR

_mesh = plsc.VectorSubcoreMesh(core_axis_name="c", subcore_axis_name="s")


def _mmT(a, b):
    return lax.dot_general(a, b, (((1,), (1,)), ((), ())),
                           preferred_element_type=jnp.float32)


def _lnk(x, g, b):
    m = jnp.mean(x, axis=-1, keepdims=True)
    v = jnp.mean((x - m) * (x - m), axis=-1, keepdims=True)
    return (x - m) / jnp.sqrt(v + 1e-5) * g + b


def _row_spec(w):
    return pl.BlockSpec((BR,) + w[1:], lambda i: (i,) + (0,) * (len(w) - 1))


def _full_spec(shape):
    return pl.BlockSpec(shape, lambda i: (0,) * len(shape))


def _tc_call(body, ins, out_shapes):
    return pl.pallas_call(
        body,
        grid=(GRID,),
        in_specs=[_row_spec(a.shape) if a.shape[0] in (N, NE, NP)
                  else _full_spec(a.shape) for a in ins],
        out_specs=[_row_spec(s) for s in out_shapes],
        out_shape=[jax.ShapeDtypeStruct(s, jnp.float32) for s in out_shapes],
    )(*ins)


def _pre_tail(X, g1, b1, w1, bw1, g2, w2, Y_o, u_o, sq_o):
    Y_o[...] = _mmT(_lnk(X, g1[...], b1[...]), w1[...]) + bw1[...]
    u_o[...] = _mmT(X, w2[...][:, :D] * g2[...][:, :D])
    su = jnp.sum(X, axis=1, keepdims=True)
    qv = jnp.sum(X * X, axis=1, keepdims=True)
    sq_o[...] = jnp.concatenate(
        [su, qv, jnp.zeros((X.shape[0], 6), jnp.float32)], axis=1)


def _head_pre_body(x_r, w_in, b_in, g1, b1, w1, bw1, g2, w2,
                   X_o, Y_o, u_o, sq_o):
    X = jnp.maximum(_mmT(x_r[...], w_in[...]) + b_in[...], 0.0)
    X_o[...] = X
    _pre_tail(X, g1, b1, w1, bw1, g2, w2, Y_o, u_o, sq_o)


def _reduce2_body(a_r, b_r, ao, bo):
    ao[...] = jnp.sum(a_r[...].reshape(NW, HR, D), axis=0)
    bo[...] = jnp.sum(b_r[...].reshape(NW, HR, D), axis=0)


def _tc_reduce2(a, b):
    a = a.reshape(NW * HR, D)
    b = b.reshape(NW * HR, D)
    return pl.pallas_call(
        _reduce2_body,
        grid=(1,),
        in_specs=[_full_spec(a.shape), _full_spec(b.shape)],
        out_specs=[_full_spec((HR, D)), _full_spec((HR, D))],
        out_shape=[jax.ShapeDtypeStruct((HR, D), jnp.float32)] * 2,
    )(a, b)


def _mid_body(t0_r, t1_r, ce_r, g2, w2, Wt_o, sqe_o):
    T = t0_r[...] + t1_r[...]
    Xe = T / jnp.maximum(ce_r[...], 1.0)
    Wt_o[...] = _mmT(Xe, w2[...][:, D:] * g2[...][:, D:])
    se = jnp.sum(Xe, axis=1, keepdims=True)
    qe = jnp.sum(Xe * Xe, axis=1, keepdims=True)
    sqe_o[...] = jnp.concatenate(
        [se, qe, jnp.zeros((Xe.shape[0], 6), jnp.float32)], axis=1)


def _post_common(g0_r, g1_r, a_r, b_r, cv_r, u_r, x0_r,
                 g2, b2, w2, bw2, g3, b3, w3, bw3):
    G = g0_r[...] + g1_r[...]
    t = jnp.sum(w2[...] * g2[...], axis=1)[None, :]
    cc = jnp.sum(w2[...] * b2[...], axis=1)[None, :] + bw2[...]
    cnt = cv_r[...]
    Xv = (a_r[...] * u_r[...] + G - b_r[...] * t
          + cnt * cc) / jnp.maximum(cnt, 1.0)
    Xn = 0.5 * Xv + 0.5 * x0_r[...]
    return jnp.maximum(
        _mmT(_lnk(Xn, g3[...], b3[...]), w3[...]) + bw3[...], 0.0)


def _post_pre_body(g0_r, g1_r, a_r, b_r, cv_r, u_r, x0_r,
                   g2, b2, w2, bw2, g3, b3, w3, bw3,
                   g1p, b1p, w1p, bw1p, Y_o, u_o, sq_o):
    X = _post_common(g0_r, g1_r, a_r, b_r, cv_r, u_r, x0_r,
                     g2, b2, w2, bw2, g3, b3, w3, bw3)
    _pre_tail(X, g1p, b1p, w1p, bw1p, g2, w2, Y_o, u_o, sq_o)


def _post_cls_body(g0_r, g1_r, a_r, b_r, cv_r, u_r, x0_r,
                   g2, b2, w2, bw2, g3, b3, w3, bw3,
                   c1, bc1, cg, cb, c2p, bc2p, o_o):
    X = _post_common(g0_r, g1_r, a_r, b_r, cv_r, u_r, x0_r,
                     g2, b2, w2, bw2, g3, b3, w3, bw3)
    h = jnp.maximum(_mmT(X, c1[...]) + bc1[...], 0.0)
    h = _lnk(h, cg[...], cb[...])
    o_o[...] = _mmT(h, c2p[...]) + bc2p[...]


KWP = 40
RING = 5
NWINP = CHUNK // KWP
ROUNDS = NWINP // RING


def _zero1d(ref, n):
    def zr(r, carry):
        ref[pl.ds(r * 16, 16)] = jnp.zeros((16,), jnp.float32)
        return carry
    lax.fori_loop(0, n // 16, zr, 0)


_IDX = pltpu.VMEM((KWP,), jnp.int32)
_ROWS = pltpu.VMEM((KWP, D), jnp.float32)
_SEM = pltpu.SemaphoreType.DMA


@functools.partial(
    pl.kernel,
    out_type=[jax.ShapeDtypeStruct((NW * NP,), jnp.float32)] * 2,
    mesh=_mesh,
    compiler_params=pltpu.CompilerParams(needs_layout_passes=False),
    scratch_types=[
        pltpu.VMEM((CHUNK,), jnp.int32),
        pltpu.VMEM((CHUNK,), jnp.int32),
        pltpu.VMEM((NP,), jnp.float32),
        pltpu.VMEM((NP,), jnp.float32),
    ],
)
def _sc_hist(v_h, e_h, oute_h, outv_h, v_t, e_t, hv, he):
    c = lax.axis_index("c")
    s = lax.axis_index("s")
    wid = c * NS + s
    base = wid * CHUNK
    pltpu.sync_copy(v_h.at[pl.ds(base, CHUNK)], v_t)
    pltpu.sync_copy(e_h.at[pl.ds(base, CHUNK)], e_t)
    _zero1d(hv, NP)
    _zero1d(he, NP)
    ones = jnp.full((16,), 1.0, jnp.float32)

    def body(g, carry):
        sl = pl.ds(g * 16, 16)
        plsc.addupdate_scatter(hv, [v_t[sl]], ones)
        plsc.addupdate_scatter(he, [e_t[sl]], ones)
        return carry

    lax.fori_loop(0, CHUNK // 16, body, 0)
    pltpu.sync_copy(he, oute_h.at[pl.ds(wid * NP, NP)])
    pltpu.sync_copy(hv, outv_h.at[pl.ds(wid * NP, NP)])


WV = 16
NWV = CHUNK // WV
VRING = 5
VROUNDS = NWV // VRING


@functools.partial(
    pl.kernel,
    out_type=jax.ShapeDtypeStruct((NC, NP, D), jnp.float32),
    mesh=_mesh,
    compiler_params=pltpu.CompilerParams(needs_layout_passes=False),
    scratch_types=[pltpu.VMEM((CHUNK,), jnp.int32)] * 2
    + [pltpu.VMEM((WV, D), jnp.float32)] * VRING + [_SEM] * (2 * VRING)
    + [pltpu.VMEM_SHARED((NP, D), jnp.float32)],
)
def _sc_gather_scatter(y_h, v_h, e_h, z_h, out_h, *refs):
    v_t, e_t = refs[0], refs[1]
    RW = refs[2:2 + VRING]
    GS = refs[2 + VRING:2 + 2 * VRING]
    SS = refs[2 + 2 * VRING:2 + 3 * VRING]
    acc = refs[2 + 3 * VRING]
    c = lax.axis_index("c")
    s = lax.axis_index("s")
    base = (c * NS + s) * CHUNK
    pltpu.sync_copy(v_h.at[pl.ds(base, CHUNK)], v_t)
    pltpu.sync_copy(e_h.at[pl.ds(base, CHUNK)], e_t)
    pltpu.sync_copy(z_h.at[pl.ds(s * RPT, RPT)], acc.at[pl.ds(s * RPT, RPT)])
    plsc.subcore_barrier()

    for j in range(VRING):
        pltpu.async_copy(y_h.at[v_t[pl.ds(j * WV, WV)]], RW[j], GS[j])

    def round_body(k, carry):
        w0 = k * VRING
        for j in range(VRING):
            sl = pl.ds((w0 + j) * WV, WV)
            pltpu.make_async_copy(y_h.at[v_t[sl]], RW[j], GS[j]).wait()
            pltpu.async_copy(RW[j], acc.at[e_t[sl]], SS[j], add=True)
        for j in range(VRING):
            sl = pl.ds((w0 + j) * WV, WV)
            nsl = pl.ds((w0 + VRING + j) * WV, WV)
            pltpu.make_async_copy(RW[j], acc.at[e_t[sl]], SS[j]).wait()
            pltpu.async_copy(y_h.at[v_t[nsl]], RW[j], GS[j])
        return carry

    lax.fori_loop(0, VROUNDS - 1, round_body, 0)
    w0 = (VROUNDS - 1) * VRING
    for j in range(VRING):
        sl = pl.ds((w0 + j) * WV, WV)
        pltpu.make_async_copy(y_h.at[v_t[sl]], RW[j], GS[j]).wait()
        pltpu.async_copy(RW[j], acc.at[e_t[sl]], SS[j], add=True)
    for j in range(VRING):
        sl = pl.ds((w0 + j) * WV, WV)
        pltpu.make_async_copy(RW[j], acc.at[e_t[sl]], SS[j]).wait()
    plsc.subcore_barrier()
    pltpu.sync_copy(acc.at[pl.ds(s * RPT, RPT)],
                    out_h.at[c, pl.ds(s * RPT, RPT)])


def _rsqrt_sc(x):
    i = plsc.bitcast(x, jnp.int32)
    i = 0x5F3759DF - lax.shift_right_logical(i, 1)
    y = plsc.bitcast(i, jnp.float32)
    for _ in range(3):
        y = y * (1.5 - 0.5 * x * y * y)
    return y


@functools.partial(
    pl.kernel,
    out_type=[jax.ShapeDtypeStruct((NNZ,), jnp.float32),
              jax.ShapeDtypeStruct((NW * NP,), jnp.float32),
              jax.ShapeDtypeStruct((NW * NP,), jnp.float32)],
    mesh=_mesh,
    compiler_params=pltpu.CompilerParams(needs_layout_passes=False),
    scratch_types=[
        pltpu.VMEM((CHUNK,), jnp.int32),
        pltpu.VMEM((CHUNK,), jnp.int32),
        pltpu.VMEM((CHUNK,), jnp.float32),
        pltpu.VMEM((N,), jnp.float32),
        pltpu.VMEM((N,), jnp.float32),
        pltpu.VMEM((NE,), jnp.float32),
        pltpu.VMEM((NE,), jnp.float32),
        pltpu.VMEM((NP,), jnp.float32),
        pltpu.VMEM((NP,), jnp.float32),
    ],
)
def _sc_stats(su_h, qv_h, se_h, qe_h, v_h, e_h, outal_h, outa_h, outb_h,
              v_t, e_t, af, su_t, qv_t, se_t, qe_t, a1d, b1d):
    c = lax.axis_index("c")
    s = lax.axis_index("s")
    wid = c * NS + s
    base = wid * CHUNK
    pltpu.sync_copy(v_h.at[pl.ds(base, CHUNK)], v_t)
    pltpu.sync_copy(e_h.at[pl.ds(base, CHUNK)], e_t)
    pltpu.sync_copy(su_h, su_t)
    pltpu.sync_copy(qv_h, qv_t)
    pltpu.sync_copy(se_h, se_t)
    pltpu.sync_copy(qe_h, qe_t)
    _zero1d(a1d, NP)
    _zero1d(b1d, NP)

    def group(sl):
        viv = v_t[sl]
        eiv = e_t[sl]
        m = (plsc.load_gather(su_t, [viv])
             + plsc.load_gather(se_t, [eiv])) * (1.0 / 256.0)
        q = (plsc.load_gather(qv_t, [viv])
             + plsc.load_gather(qe_t, [eiv])) * (1.0 / 256.0)
        a = _rsqrt_sc(q - m * m + 1e-5)
        af[sl] = a
        plsc.addupdate_scatter(a1d, [viv], a)
        plsc.addupdate_scatter(b1d, [viv], m * a)

    def body(g2, carry):
        for u in range(2):
            group(pl.ds(g2 * 32 + u * 16, 16))
        return carry

    lax.fori_loop(0, CHUNK // 32, body, 0)
    for gt in range(CHUNK // 32 * 2, CHUNK // 16):
        group(pl.ds(gt * 16, 16))
    pltpu.sync_copy(af, outal_h.at[pl.ds(base, CHUNK)])
    pltpu.sync_copy(a1d, outa_h.at[pl.ds(wid * NP, NP)])
    pltpu.sync_copy(b1d, outb_h.at[pl.ds(wid * NP, NP)])


@functools.partial(
    pl.kernel,
    out_type=jax.ShapeDtypeStruct((NC, NP, D), jnp.float32),
    mesh=_mesh,
    compiler_params=pltpu.CompilerParams(needs_layout_passes=False),
    scratch_types=[pltpu.VMEM((CHUNK,), jnp.int32)] * 2
    + [pltpu.VMEM((CHUNK,), jnp.float32)]
    + [pltpu.VMEM((WV, D), jnp.float32)] * VRING + [_SEM] * (2 * VRING)
    + [pltpu.VMEM_SHARED((NP, D), jnp.float32)],
)
def _sc_weighted_scatter(w_hbm, al_h, v_h, e_h, z_h, out_h, *refs):
    v_t, e_t, af = refs[0], refs[1], refs[2]
    RW = refs[3:3 + VRING]
    GS = refs[3 + VRING:3 + 2 * VRING]
    SS = refs[3 + 2 * VRING:3 + 3 * VRING]
    acc = refs[3 + 3 * VRING]
    c = lax.axis_index("c")
    s = lax.axis_index("s")
    base = (c * NS + s) * CHUNK
    pltpu.sync_copy(v_h.at[pl.ds(base, CHUNK)], v_t)
    pltpu.sync_copy(e_h.at[pl.ds(base, CHUNK)], e_t)
    pltpu.sync_copy(al_h.at[pl.ds(base, CHUNK)], af)
    pltpu.sync_copy(z_h.at[pl.ds(s * RPT, RPT)], acc.at[pl.ds(s * RPT, RPT)])
    plsc.subcore_barrier()

    for j in range(VRING):
        pltpu.async_copy(w_hbm.at[e_t[pl.ds(j * WV, WV)]], RW[j], GS[j])

    def scale_rows(rows_ref, w):
        for r in range(WV):
            av = plsc.load_gather(
                af, [jnp.full((16,), w * WV + r, jnp.int32)])
            for cc in range(D // 16):
                csl = pl.ds(cc * 16, 16)
                rows_ref[r, csl] = rows_ref[r, csl] * av

    def round_body(k, carry):
        w0 = k * VRING
        for j in range(VRING):
            sl = pl.ds((w0 + j) * WV, WV)
            pltpu.make_async_copy(w_hbm.at[e_t[sl]], RW[j], GS[j]).wait()
            scale_rows(RW[j], w0 + j)
            pltpu.async_copy(RW[j], acc.at[v_t[sl]], SS[j], add=True)
        for j in range(VRING):
            sl = pl.ds((w0 + j) * WV, WV)
            nsl = pl.ds((w0 + VRING + j) * WV, WV)
            pltpu.make_async_copy(RW[j], acc.at[v_t[sl]], SS[j]).wait()
            pltpu.async_copy(w_hbm.at[e_t[nsl]], RW[j], GS[j])
        return carry

    lax.fori_loop(0, VROUNDS - 1, round_body, 0)
    w0 = (VROUNDS - 1) * VRING
    for j in range(VRING):
        sl = pl.ds((w0 + j) * WV, WV)
        pltpu.make_async_copy(w_hbm.at[e_t[sl]], RW[j], GS[j]).wait()
        scale_rows(RW[j], w0 + j)
        pltpu.async_copy(RW[j], acc.at[v_t[sl]], SS[j], add=True)
    for j in range(VRING):
        sl = pl.ds((w0 + j) * WV, WV)
        pltpu.make_async_copy(RW[j], acc.at[v_t[sl]], SS[j]).wait()
    plsc.subcore_barrier()
    pltpu.sync_copy(acc.at[pl.ds(s * RPT, RPT)],
                    out_h.at[c, pl.ds(s * RPT, RPT)])


def _col(h2d):
    return h2d.reshape(NP)[:N].reshape(N, 1)


def _branch(x, v, e, zeros, p):
    (w_in, b_in, g1, b1, w1, bw1, g2, b2, w2, bw2, g3, b3, w3, bw3,
     c1, bc1, cg, cb, c2p, bc2p) = p
    X0, Y, u, sq = _tc_call(
        _head_pre_body, [x, w_in, b_in, g1, b1, w1, bw1, g2, w2],
        [(N, D), (N, D), (N, D), (N, 8)])
    he, hv = _sc_hist(v, e)
    ce2, cv2 = _tc_reduce2(he, hv)
    cnt_e = _col(ce2)
    cnt_v = _col(cv2)
    for conv in range(2):
        Tp = _sc_gather_scatter(Y, v, e, zeros)
        Wt, sqe = _tc_call(_mid_body, [Tp[0], Tp[1], cnt_e, g2, w2],
                           [(NE, D), (NE, 8)])
        su = sq[:, 0] + 0.0
        qv = sq[:, 1] + 0.0
        se = sqe[:, 0] + 0.0
        qe = sqe[:, 1] + 0.0
        alv, Ap, Bp = _sc_stats(su, qv, se, qe, v, e)
        Gp = _sc_weighted_scatter(Wt, alv, v, e, zeros)
        A2, B2 = _tc_reduce2(Ap, Bp)
        A = _col(A2)
        B = _col(B2)
        if conv == 0:
            Y, u, sq = _tc_call(
                _post_pre_body,
                [Gp[0], Gp[1], A, B, cnt_v, u, X0, g2, b2, w2, bw2,
                 g3, b3, w3, bw3, g1, b1, w1, bw1],
                [(N, D), (N, D), (N, 8)])
        else:
            o8, = _tc_call(
                _post_cls_body,
                [Gp[0], Gp[1], A, B, cnt_v, u, X0, g2, b2, w2, bw2,
                 g3, b3, w3, bw3, c1, bc1, cg, cb, c2p, bc2p],
                [(N, 8)])
    return o8[:, :1]


def kernel(x1, v1, e1, x2, v2, e2, w_in, b_in, ln1_g, ln1_b, w1, bw1,
           ln2_g, ln2_b, w2, bw2, ln3_g, ln3_b, w3, bw3, c1, bc1,
           cln_g, cln_b, c2, bc2):
    r1 = lambda a: a.reshape(1, -1)
    c2p = jnp.concatenate([c2, jnp.zeros((7, 64), jnp.float32)], axis=0)
    bc2p = jnp.concatenate([bc2, jnp.zeros((7,), jnp.float32)]).reshape(1, 8)
    p = (w_in, r1(b_in), r1(ln1_g), r1(ln1_b), w1, r1(bw1), r1(ln2_g),
         r1(ln2_b), w2, r1(bw2), r1(ln3_g), r1(ln3_b), w3, r1(bw3),
         c1, r1(bc1), r1(cln_g), r1(cln_b), c2p, bc2p)
    zeros = jnp.zeros((NP, D), jnp.float32)
    o1 = _branch(x1, v1, e1, zeros, p)
    o2 = _branch(x2, v2, e2, zeros, p)
    return (o1, o2)


# speedup vs baseline: 1.1515x; 1.1515x over previous
import functools

import jax
import jax.numpy as jnp
from jax import lax
from jax.experimental import pallas as pl
from jax.experimental.pallas import tpu as pltpu
from jax.experimental.pallas import tpu_sc as plsc

N = 10000
NE = 10000
NNZ = 320000
D = 128
NC = 2
NS = 16
NW = NC * NS
CHUNK = NNZ // NW
KW = 80
NWIN = CHUNK // KW
NP = 10240
RPT = NP // NS
HR = NP // D

BR = 1000
GRID = N // BR

_mesh = plsc.VectorSubcoreMesh(core_axis_name="c", subcore_axis_name="s")


def _mmT(a, b):
    return lax.dot_general(a, b, (((1,), (1,)), ((), ())),
                           preferred_element_type=jnp.float32)


def _lnk(x, g, b):
    m = jnp.mean(x, axis=-1, keepdims=True)
    v = jnp.mean((x - m) * (x - m), axis=-1, keepdims=True)
    return (x - m) / jnp.sqrt(v + 1e-5) * g + b


def _row_spec(w):
    return pl.BlockSpec((BR,) + w[1:], lambda i: (i,) + (0,) * (len(w) - 1))


def _full_spec(shape):
    return pl.BlockSpec(shape, lambda i: (0,) * len(shape))


def _tc_call(body, ins, out_shapes):
    return pl.pallas_call(
        body,
        grid=(GRID,),
        in_specs=[_row_spec(a.shape) if a.shape[0] in (N, NE, NP)
                  else _full_spec(a.shape) for a in ins],
        out_specs=[_row_spec(s) for s in out_shapes],
        out_shape=[jax.ShapeDtypeStruct(s, jnp.float32) for s in out_shapes],
    )(*ins)


def _pre_tail(X, g1, b1, w1, bw1, g2, w2, Y_o, u_o, sq_o):
    Y_o[...] = _mmT(_lnk(X, g1[...], b1[...]), w1[...]) + bw1[...]
    u_o[...] = _mmT(X, w2[...][:, :D] * g2[...][:, :D])
    su = jnp.sum(X, axis=1, keepdims=True)
    qv = jnp.sum(X * X, axis=1, keepdims=True)
    sq_o[...] = jnp.concatenate(
        [su, qv, jnp.zeros((X.shape[0], 6), jnp.float32)], axis=1)


def _head_pre_body(x_r, w_in, b_in, g1, b1, w1, bw1, g2, w2,
                   X_o, Y_o, u_o, sq_o):
    X = jnp.maximum(_mmT(x_r[...], w_in[...]) + b_in[...], 0.0)
    X_o[...] = X
    _pre_tail(X, g1, b1, w1, bw1, g2, w2, Y_o, u_o, sq_o)


def _reduce2_body(a_r, b_r, ao, bo):
    ao[...] = jnp.sum(a_r[...].reshape(NW, HR, D), axis=0)
    bo[...] = jnp.sum(b_r[...].reshape(NW, HR, D), axis=0)


def _tc_reduce2(a, b):
    a = a.reshape(NW * HR, D)
    b = b.reshape(NW * HR, D)
    return pl.pallas_call(
        _reduce2_body,
        grid=(1,),
        in_specs=[_full_spec(a.shape), _full_spec(b.shape)],
        out_specs=[_full_spec((HR, D)), _full_spec((HR, D))],
        out_shape=[jax.ShapeDtypeStruct((HR, D), jnp.float32)] * 2,
    )(a, b)


def _mid_body(t0_r, t1_r, ce_r, g2, w2, Wt_o, sqe_o):
    T = t0_r[...] + t1_r[...]
    Xe = T / jnp.maximum(ce_r[...], 1.0)
    Wt_o[...] = _mmT(Xe, w2[...][:, D:] * g2[...][:, D:])
    se = jnp.sum(Xe, axis=1, keepdims=True)
    qe = jnp.sum(Xe * Xe, axis=1, keepdims=True)
    sqe_o[...] = jnp.concatenate(
        [se, qe, jnp.zeros((Xe.shape[0], 6), jnp.float32)], axis=1)


def _post_common(g0_r, g1_r, a_r, b_r, cv_r, u_r, x0_r,
                 g2, b2, w2, bw2, g3, b3, w3, bw3):
    G = g0_r[...] + g1_r[...]
    t = jnp.sum(w2[...] * g2[...], axis=1)[None, :]
    cc = jnp.sum(w2[...] * b2[...], axis=1)[None, :] + bw2[...]
    cnt = cv_r[...]
    Xv = (a_r[...] * u_r[...] + G - b_r[...] * t
          + cnt * cc) / jnp.maximum(cnt, 1.0)
    Xn = 0.5 * Xv + 0.5 * x0_r[...]
    return jnp.maximum(
        _mmT(_lnk(Xn, g3[...], b3[...]), w3[...]) + bw3[...], 0.0)


def _post_pre_body(g0_r, g1_r, a_r, b_r, cv_r, u_r, x0_r,
                   g2, b2, w2, bw2, g3, b3, w3, bw3,
                   g1p, b1p, w1p, bw1p, Y_o, u_o, sq_o):
    X = _post_common(g0_r, g1_r, a_r, b_r, cv_r, u_r, x0_r,
                     g2, b2, w2, bw2, g3, b3, w3, bw3)
    _pre_tail(X, g1p, b1p, w1p, bw1p, g2, w2, Y_o, u_o, sq_o)


def _post_cls_body(g0_r, g1_r, a_r, b_r, cv_r, u_r, x0_r,
                   g2, b2, w2, bw2, g3, b3, w3, bw3,
                   c1, bc1, cg, cb, c2p, bc2p, o_o):
    X = _post_common(g0_r, g1_r, a_r, b_r, cv_r, u_r, x0_r,
                     g2, b2, w2, bw2, g3, b3, w3, bw3)
    h = jnp.maximum(_mmT(X, c1[...]) + bc1[...], 0.0)
    h = _lnk(h, cg[...], cb[...])
    o_o[...] = _mmT(h, c2p[...]) + bc2p[...]


KWP = 40
RING = 5
NWINP = CHUNK // KWP
ROUNDS = NWINP // RING


def _zero1d(ref, n):
    def zr(r, carry):
        ref[pl.ds(r * 16, 16)] = jnp.zeros((16,), jnp.float32)
        return carry
    lax.fori_loop(0, n // 16, zr, 0)


_IDX = pltpu.VMEM((KWP,), jnp.int32)
_ROWS = pltpu.VMEM((KWP, D), jnp.float32)
_SEM = pltpu.SemaphoreType.DMA


@functools.partial(
    pl.kernel,
    out_type=[jax.ShapeDtypeStruct((NW * NP,), jnp.float32)] * 2,
    mesh=_mesh,
    compiler_params=pltpu.CompilerParams(needs_layout_passes=False),
    scratch_types=[
        pltpu.VMEM((CHUNK,), jnp.int32),
        pltpu.VMEM((CHUNK,), jnp.int32),
        pltpu.VMEM((NP,), jnp.float32),
        pltpu.VMEM((NP,), jnp.float32),
    ],
)
def _sc_hist(v_h, e_h, oute_h, outv_h, v_t, e_t, hv, he):
    c = lax.axis_index("c")
    s = lax.axis_index("s")
    wid = c * NS + s
    base = wid * CHUNK
    pltpu.sync_copy(v_h.at[pl.ds(base, CHUNK)], v_t)
    pltpu.sync_copy(e_h.at[pl.ds(base, CHUNK)], e_t)
    _zero1d(hv, NP)
    _zero1d(he, NP)
    ones = jnp.full((16,), 1.0, jnp.float32)

    def body(g, carry):
        sl = pl.ds(g * 16, 16)
        plsc.addupdate_scatter(hv, [v_t[sl]], ones)
        plsc.addupdate_scatter(he, [e_t[sl]], ones)
        return carry

    lax.fori_loop(0, CHUNK // 16, body, 0)
    pltpu.sync_copy(he, oute_h.at[pl.ds(wid * NP, NP)])
    pltpu.sync_copy(hv, outv_h.at[pl.ds(wid * NP, NP)])


@functools.partial(
    pl.kernel,
    out_type=jax.ShapeDtypeStruct((NC, NP, D), jnp.float32),
    mesh=_mesh,
    compiler_params=pltpu.CompilerParams(needs_layout_passes=False),
    scratch_types=[_IDX] * RING + [_IDX] * RING + [_ROWS] * RING
    + [pltpu.VMEM_SHARED((NP, D), jnp.float32)] + [_SEM] * (3 * RING),
)
def _sc_gather_scatter(y_h, v_h, e_h, z_h, out_h, *refs):
    VI = refs[0:RING]
    EI = refs[RING:2 * RING]
    RW = refs[2 * RING:3 * RING]
    acc = refs[3 * RING]
    IS = refs[3 * RING + 1:4 * RING + 1]
    GS = refs[4 * RING + 1:5 * RING + 1]
    SS = refs[5 * RING + 1:6 * RING + 1]
    c = lax.axis_index("c")
    s = lax.axis_index("s")
    base = (c * NS + s) * CHUNK
    pltpu.sync_copy(z_h.at[pl.ds(s * RPT, RPT)], acc.at[pl.ds(s * RPT, RPT)])
    plsc.subcore_barrier()

    for j in range(RING):
        off = base + j * KWP
        iv = pltpu.async_copy(v_h.at[pl.ds(off, KWP)], VI[j], IS[j])
        ie = pltpu.async_copy(e_h.at[pl.ds(off, KWP)], EI[j], IS[j])
        iv.wait()
        ie.wait()
        pltpu.async_copy(y_h.at[VI[j]], RW[j], GS[j])

    def round_body(k, carry):
        for j in range(RING):
            pltpu.make_async_copy(y_h.at[VI[j]], RW[j], GS[j]).wait()
            pltpu.async_copy(RW[j], acc.at[EI[j]], SS[j], add=True)
        for j in range(RING):
            pltpu.make_async_copy(RW[j], acc.at[EI[j]], SS[j]).wait()
            off = base + ((k + 1) * RING + j) * KWP
            iv = pltpu.async_copy(v_h.at[pl.ds(off, KWP)], VI[j], IS[j])
            ie = pltpu.async_copy(e_h.at[pl.ds(off, KWP)], EI[j], IS[j])
            iv.wait()
            ie.wait()
            pltpu.async_copy(y_h.at[VI[j]], RW[j], GS[j])
        return carry

    lax.fori_loop(0, ROUNDS - 1, round_body, 0)
    for j in range(RING):
        pltpu.make_async_copy(y_h.at[VI[j]], RW[j], GS[j]).wait()
        pltpu.async_copy(RW[j], acc.at[EI[j]], SS[j], add=True)
    for j in range(RING):
        pltpu.make_async_copy(RW[j], acc.at[EI[j]], SS[j]).wait()
    plsc.subcore_barrier()
    pltpu.sync_copy(acc.at[pl.ds(s * RPT, RPT)],
                    out_h.at[c, pl.ds(s * RPT, RPT)])


def _rsqrt_sc(x):
    i = plsc.bitcast(x, jnp.int32)
    i = 0x5F3759DF - lax.shift_right_logical(i, 1)
    y = plsc.bitcast(i, jnp.float32)
    for _ in range(3):
        y = y * (1.5 - 0.5 * x * y * y)
    return y


@functools.partial(
    pl.kernel,
    out_type=[jax.ShapeDtypeStruct((NNZ,), jnp.float32),
              jax.ShapeDtypeStruct((NW * NP,), jnp.float32),
              jax.ShapeDtypeStruct((NW * NP,), jnp.float32)],
    mesh=_mesh,
    compiler_params=pltpu.CompilerParams(needs_layout_passes=False),
    scratch_types=[
        pltpu.VMEM((CHUNK,), jnp.int32),
        pltpu.VMEM((CHUNK,), jnp.int32),
        pltpu.VMEM((CHUNK,), jnp.float32),
        pltpu.VMEM((N,), jnp.float32),
        pltpu.VMEM((N,), jnp.float32),
        pltpu.VMEM((NE,), jnp.float32),
        pltpu.VMEM((NE,), jnp.float32),
        pltpu.VMEM((NP,), jnp.float32),
        pltpu.VMEM((NP,), jnp.float32),
    ],
)
def _sc_stats(su_h, qv_h, se_h, qe_h, v_h, e_h, outal_h, outa_h, outb_h,
              v_t, e_t, af, su_t, qv_t, se_t, qe_t, a1d, b1d):
    c = lax.axis_index("c")
    s = lax.axis_index("s")
    wid = c * NS + s
    base = wid * CHUNK
    pltpu.sync_copy(v_h.at[pl.ds(base, CHUNK)], v_t)
    pltpu.sync_copy(e_h.at[pl.ds(base, CHUNK)], e_t)
    pltpu.sync_copy(su_h, su_t)
    pltpu.sync_copy(qv_h, qv_t)
    pltpu.sync_copy(se_h, se_t)
    pltpu.sync_copy(qe_h, qe_t)
    _zero1d(a1d, NP)
    _zero1d(b1d, NP)

    def group(sl):
        viv = v_t[sl]
        eiv = e_t[sl]
        m = (plsc.load_gather(su_t, [viv])
             + plsc.load_gather(se_t, [eiv])) * (1.0 / 256.0)
        q = (plsc.load_gather(qv_t, [viv])
             + plsc.load_gather(qe_t, [eiv])) * (1.0 / 256.0)
        a = _rsqrt_sc(q - m * m + 1e-5)
        af[sl] = a
        plsc.addupdate_scatter(a1d, [viv], a)
        plsc.addupdate_scatter(b1d, [viv], m * a)

    def body(g2, carry):
        for u in range(2):
            group(pl.ds(g2 * 32 + u * 16, 16))
        return carry

    lax.fori_loop(0, CHUNK // 32, body, 0)
    for gt in range(CHUNK // 32 * 2, CHUNK // 16):
        group(pl.ds(gt * 16, 16))
    pltpu.sync_copy(af, outal_h.at[pl.ds(base, CHUNK)])
    pltpu.sync_copy(a1d, outa_h.at[pl.ds(wid * NP, NP)])
    pltpu.sync_copy(b1d, outb_h.at[pl.ds(wid * NP, NP)])


@functools.partial(
    pl.kernel,
    out_type=jax.ShapeDtypeStruct((NC, NP, D), jnp.float32),
    mesh=_mesh,
    compiler_params=pltpu.CompilerParams(needs_layout_passes=False),
    scratch_types=[_IDX] * RING + [_IDX] * RING + [_ROWS] * RING
    + [pltpu.VMEM((CHUNK,), jnp.float32)]
    + [pltpu.VMEM_SHARED((NP, D), jnp.float32)] + [_SEM] * (3 * RING),
)
def _sc_weighted_scatter(w_hbm, al_h, v_h, e_h, z_h, out_h, *refs):
    VI = refs[0:RING]
    EI = refs[RING:2 * RING]
    RW = refs[2 * RING:3 * RING]
    af = refs[3 * RING]
    acc = refs[3 * RING + 1]
    IS = refs[3 * RING + 2:4 * RING + 2]
    GS = refs[4 * RING + 2:5 * RING + 2]
    SS = refs[5 * RING + 2:6 * RING + 2]
    c = lax.axis_index("c")
    s = lax.axis_index("s")
    base = (c * NS + s) * CHUNK
    pltpu.sync_copy(al_h.at[pl.ds(base, CHUNK)], af)
    pltpu.sync_copy(z_h.at[pl.ds(s * RPT, RPT)], acc.at[pl.ds(s * RPT, RPT)])
    plsc.subcore_barrier()

    for j in range(RING):
        off = base + j * KWP
        iv = pltpu.async_copy(v_h.at[pl.ds(off, KWP)], VI[j], IS[j])
        ie = pltpu.async_copy(e_h.at[pl.ds(off, KWP)], EI[j], IS[j])
        iv.wait()
        ie.wait()
        pltpu.async_copy(w_hbm.at[EI[j]], RW[j], GS[j])

    def scale_rows(rows_ref, w_in_chunk):
        off_a = w_in_chunk * KWP

        def rbody(r, carry2):
            av = plsc.load_gather(af, [jnp.full((16,), off_a + r, jnp.int32)])
            for cc in range(D // 16):
                csl = pl.ds(cc * 16, 16)
                rows_ref[r, csl] = rows_ref[r, csl] * av
            return carry2

        lax.fori_loop(0, KWP, rbody, 0)

    def round_body(k, carry):
        for j in range(RING):
            pltpu.make_async_copy(w_hbm.at[EI[j]], RW[j], GS[j]).wait()
            scale_rows(RW[j], k * RING + j)
            pltpu.async_copy(RW[j], acc.at[VI[j]], SS[j], add=True)
        for j in range(RING):
            pltpu.make_async_copy(RW[j], acc.at[VI[j]], SS[j]).wait()
            off = base + ((k + 1) * RING + j) * KWP
            iv = pltpu.async_copy(v_h.at[pl.ds(off, KWP)], VI[j], IS[j])
            ie = pltpu.async_copy(e_h.at[pl.ds(off, KWP)], EI[j], IS[j])
            iv.wait()
            ie.wait()
            pltpu.async_copy(w_hbm.at[EI[j]], RW[j], GS[j])
        return carry

    lax.fori_loop(0, ROUNDS - 1, round_body, 0)
    for j in range(RING):
        pltpu.make_async_copy(w_hbm.at[EI[j]], RW[j], GS[j]).wait()
        scale_rows(RW[j], (ROUNDS - 1) * RING + j)
        pltpu.async_copy(RW[j], acc.at[VI[j]], SS[j], add=True)
    for j in range(RING):
        pltpu.make_async_copy(RW[j], acc.at[VI[j]], SS[j]).wait()
    plsc.subcore_barrier()
    pltpu.sync_copy(acc.at[pl.ds(s * RPT, RPT)],
                    out_h.at[c, pl.ds(s * RPT, RPT)])


def _col(h2d):
    return h2d.reshape(NP)[:N].reshape(N, 1)


def _branch(x, v, e, zeros, p):
    (w_in, b_in, g1, b1, w1, bw1, g2, b2, w2, bw2, g3, b3, w3, bw3,
     c1, bc1, cg, cb, c2p, bc2p) = p
    X0, Y, u, sq = _tc_call(
        _head_pre_body, [x, w_in, b_in, g1, b1, w1, bw1, g2, w2],
        [(N, D), (N, D), (N, D), (N, 8)])
    he, hv = _sc_hist(v, e)
    ce2, cv2 = _tc_reduce2(he, hv)
    cnt_e = _col(ce2)
    cnt_v = _col(cv2)
    for conv in range(2):
        Tp = _sc_gather_scatter(Y, v, e, zeros)
        Wt, sqe = _tc_call(_mid_body, [Tp[0], Tp[1], cnt_e, g2, w2],
                           [(NE, D), (NE, 8)])
        su = sq[:, 0] + 0.0
        qv = sq[:, 1] + 0.0
        se = sqe[:, 0] + 0.0
        qe = sqe[:, 1] + 0.0
        alv, Ap, Bp = _sc_stats(su, qv, se, qe, v, e)
        Gp = _sc_weighted_scatter(Wt, alv, v, e, zeros)
        A2, B2 = _tc_reduce2(Ap, Bp)
        A = _col(A2)
        B = _col(B2)
        if conv == 0:
            Y, u, sq = _tc_call(
                _post_pre_body,
                [Gp[0], Gp[1], A, B, cnt_v, u, X0, g2, b2, w2, bw2,
                 g3, b3, w3, bw3, g1, b1, w1, bw1],
                [(N, D), (N, D), (N, 8)])
        else:
            o8, = _tc_call(
                _post_cls_body,
                [Gp[0], Gp[1], A, B, cnt_v, u, X0, g2, b2, w2, bw2,
                 g3, b3, w3, bw3, c1, bc1, cg, cb, c2p, bc2p],
                [(N, 8)])
    return o8[:, :1]


def kernel(x1, v1, e1, x2, v2, e2, w_in, b_in, ln1_g, ln1_b, w1, bw1,
           ln2_g, ln2_b, w2, bw2, ln3_g, ln3_b, w3, bw3, c1, bc1,
           cln_g, cln_b, c2, bc2):
    r1 = lambda a: a.reshape(1, -1)
    c2p = jnp.concatenate([c2, jnp.zeros((7, 64), jnp.float32)], axis=0)
    bc2p = jnp.concatenate([bc2, jnp.zeros((7,), jnp.float32)]).reshape(1, 8)
    p = (w_in, r1(b_in), r1(ln1_g), r1(ln1_b), w1, r1(bw1), r1(ln2_g),
         r1(ln2_b), w2, r1(bw2), r1(ln3_g), r1(ln3_b), w3, r1(bw3),
         c1, r1(bc1), r1(cln_g), r1(cln_b), c2p, bc2p)
    zeros = jnp.zeros((NP, D), jnp.float32)
    o1 = _branch(x1, v1, e1, zeros, p)
    o2 = _branch(x2, v2, e2, zeros, p)
    return (o1, o2)
